# baseline (speedup 1.0000x reference)
import functools

import jax
import jax.numpy as jnp
from jax import lax
from jax.experimental import pallas as pl
from jax.experimental.pallas import tpu as pltpu
from jax.experimental.pallas import tpu_sc as plsc

BATCH = 16384
INPUT_DIM = 128
EMBED_DIM = 64

_INFO = plsc.get_sparse_core_info()
_NC, _NS = _INFO.num_cores, _INFO.num_subcores
_NW = _NC * _NS
_B_PER_W = BATCH // _NW


def _sc_gather(domain_ids, domain_table):
    mesh = plsc.VectorSubcoreMesh(core_axis_name="c", subcore_axis_name="s")

    @functools.partial(
        pl.kernel,
        mesh=mesh,
        out_type=jax.ShapeDtypeStruct((BATCH, EMBED_DIM), jnp.float32),
        scratch_types=[
            pltpu.VMEM((_B_PER_W,), jnp.int32),
            pltpu.VMEM((_B_PER_W, EMBED_DIM), jnp.float32),
            pltpu.SemaphoreType.DMA,
        ],
        compiler_params=pltpu.CompilerParams(use_tc_tiling_on_sc=False),
    )
    def gather_kernel(ids_hbm, table_hbm, out_hbm, ids_v, rows_v, sem):
        wid = lax.axis_index("s") * _NC + lax.axis_index("c")
        base = wid * _B_PER_W
        pltpu.sync_copy(ids_hbm.at[pl.ds(base, _B_PER_W)], ids_v)
        pltpu.async_copy(table_hbm.at[ids_v], rows_v, sem).wait()
        pltpu.sync_copy(rows_v, out_hbm.at[pl.ds(base, _B_PER_W)])

    return gather_kernel(domain_ids, domain_table)


_BM = 2048


def _tc_body(x_ref, w_ref, b_ref, g_ref, o_ref):
    proj = lax.dot_general(
        x_ref[...], w_ref[...],
        dimension_numbers=(((1,), (1,)), ((), ())),
        preferred_element_type=jnp.float32,
    )
    o_ref[...] = proj + b_ref[...] + g_ref[...]


def _tc_proj_add(x, W, b, g):
    grid = (BATCH // _BM,)
    return pl.pallas_call(
        _tc_body,
        grid=grid,
        in_specs=[
            pl.BlockSpec((_BM, INPUT_DIM), lambda i: (i, 0)),
            pl.BlockSpec((EMBED_DIM, INPUT_DIM), lambda i: (0, 0)),
            pl.BlockSpec((1, EMBED_DIM), lambda i: (0, 0)),
            pl.BlockSpec((_BM, EMBED_DIM), lambda i: (i, 0)),
        ],
        out_specs=pl.BlockSpec((_BM, EMBED_DIM), lambda i: (i, 0)),
        out_shape=jax.ShapeDtypeStruct((BATCH, EMBED_DIM), jnp.float32),
    )(x, W, b, g)


@jax.jit
def kernel(x, domain_ids, domain_table, W, b):
    g = _sc_gather(domain_ids.astype(jnp.int32), domain_table)
    return _tc_proj_add(x, W, b.reshape(1, EMBED_DIM), g)


# speedup vs baseline: 1.9510x; 1.9510x over previous
import functools

import jax
import jax.numpy as jnp
from jax import lax
from jax.experimental import pallas as pl
from jax.experimental.pallas import tpu as pltpu
from jax.experimental.pallas import tpu_sc as plsc

BATCH = 16384
INPUT_DIM = 128
EMBED_DIM = 64
NUM_DOMAINS = 100000

_INFO = plsc.get_sparse_core_info()
_NC, _NS = _INFO.num_cores, _INFO.num_subcores
_NW = _NC * _NS
_DIMS_PER_W = EMBED_DIM // _NW
_CHUNK = 8192
_UNROLL = 8


def _sc_gather_t(domain_ids, table_t):
    mesh = plsc.VectorSubcoreMesh(core_axis_name="c", subcore_axis_name="s")

    @functools.partial(
        pl.kernel,
        mesh=mesh,
        out_type=jax.ShapeDtypeStruct((EMBED_DIM, BATCH), jnp.float32),
        scratch_types=[
            pltpu.VMEM((NUM_DOMAINS,), jnp.float32),
            pltpu.VMEM((_CHUNK,), jnp.int32),
            pltpu.VMEM((_CHUNK,), jnp.float32),
        ],
        compiler_params=pltpu.CompilerParams(needs_layout_passes=False),
    )
    def gather_kernel(ids_hbm, table_hbm, out_hbm, slice_v, ids_v, out_v):
        wid = lax.axis_index("s") * _NC + lax.axis_index("c")

        def one_dim(d):
            j = wid * _DIMS_PER_W + d
            pltpu.sync_copy(table_hbm.at[j], slice_v)
            for cb in range(BATCH // _CHUNK):
                pltpu.sync_copy(ids_hbm.at[pl.ds(cb * _CHUNK, _CHUNK)], ids_v)

                def body(c, _):
                    base = c * (16 * _UNROLL)
                    for u in range(_UNROLL):
                        off = base + u * 16
                        idx = ids_v[pl.ds(off, 16)]
                        out_v[pl.ds(off, 16)] = plsc.load_gather(
                            slice_v, [idx]
                        )
                    return _

                lax.fori_loop(0, _CHUNK // (16 * _UNROLL), body, None)
                pltpu.sync_copy(
                    out_v, out_hbm.at[j, pl.ds(cb * _CHUNK, _CHUNK)]
                )

        for d in range(_DIMS_PER_W):
            one_dim(d)

    return gather_kernel(domain_ids, table_t)


_BN = 2048


def _tc_body(w_ref, x_ref, b_ref, g_ref, o_ref):
    proj = lax.dot_general(
        w_ref[...], x_ref[...],
        dimension_numbers=(((1,), (1,)), ((), ())),
        preferred_element_type=jnp.float32,
    )
    o_ref[...] = proj + b_ref[...] + g_ref[...]


def _tc_proj_add(W, x, b, g_t):
    grid = (BATCH // _BN,)
    return pl.pallas_call(
        _tc_body,
        grid=grid,
        in_specs=[
            pl.BlockSpec((EMBED_DIM, INPUT_DIM), lambda i: (0, 0)),
            pl.BlockSpec((_BN, INPUT_DIM), lambda i: (i, 0)),
            pl.BlockSpec((EMBED_DIM, 1), lambda i: (0, 0)),
            pl.BlockSpec((EMBED_DIM, _BN), lambda i: (0, i)),
        ],
        out_specs=pl.BlockSpec((EMBED_DIM, _BN), lambda i: (0, i)),
        out_shape=jax.ShapeDtypeStruct((EMBED_DIM, BATCH), jnp.float32),
    )(W, x, b, g_t)


@jax.jit
def kernel(x, domain_ids, domain_table, W, b):
    g_t = _sc_gather_t(domain_ids.astype(jnp.int32), domain_table.T)
    out_t = _tc_proj_add(W, x, b.reshape(EMBED_DIM, 1), g_t)
    return out_t.T


# speedup vs baseline: 2.1617x; 1.1080x over previous
import functools

import jax
import jax.numpy as jnp
from jax import lax
from jax.experimental import pallas as pl
from jax.experimental.pallas import tpu as pltpu
from jax.experimental.pallas import tpu_sc as plsc

BATCH = 16384
INPUT_DIM = 128
EMBED_DIM = 64
NUM_DOMAINS = 100000

_INFO = plsc.get_sparse_core_info()
_NC, _NS = _INFO.num_cores, _INFO.num_subcores
_NW = _NC * _NS
_DIMS_PER_W = EMBED_DIM // _NW
_CHUNK = 8192
_UNROLL = 8


def _sc_gather_t(domain_ids, table_t):
    mesh = plsc.VectorSubcoreMesh(core_axis_name="c", subcore_axis_name="s")

    @functools.partial(
        pl.kernel,
        mesh=mesh,
        out_type=jax.ShapeDtypeStruct((EMBED_DIM, BATCH), jnp.float32),
        scratch_types=[
            pltpu.VMEM((NUM_DOMAINS,), jnp.float32),
            pltpu.VMEM((_CHUNK,), jnp.int32),
            pltpu.VMEM((_CHUNK,), jnp.float32),
        ],
        compiler_params=pltpu.CompilerParams(needs_layout_passes=False),
    )
    def gather_kernel(ids_hbm, table_hbm, out_hbm, slice_v, ids_v, out_v):
        wid = lax.axis_index("s") * _NC + lax.axis_index("c")

        def one_dim(d):
            j = wid * _DIMS_PER_W + d
            pltpu.sync_copy(table_hbm.at[j], slice_v)
            for cb in range(BATCH // _CHUNK):
                pltpu.sync_copy(ids_hbm.at[pl.ds(cb * _CHUNK, _CHUNK)], ids_v)

                @plsc.parallel_loop(0, _CHUNK // 16, unroll=_UNROLL)
                def body(c):
                    off = c * 16
                    idx = ids_v[pl.ds(off, 16)]
                    out_v[pl.ds(off, 16)] = plsc.load_gather(slice_v, [idx])

                pltpu.sync_copy(
                    out_v, out_hbm.at[j, pl.ds(cb * _CHUNK, _CHUNK)]
                )

        for d in range(_DIMS_PER_W):
            one_dim(d)

    return gather_kernel(domain_ids, table_t)


_BN = 2048


def _tc_body(w_ref, x_ref, b_ref, g_ref, o_ref):
    proj = lax.dot_general(
        w_ref[...], x_ref[...],
        dimension_numbers=(((1,), (1,)), ((), ())),
        preferred_element_type=jnp.float32,
    )
    o_ref[...] = proj + b_ref[...] + g_ref[...]


def _tc_proj_add(W, x, b, g_t):
    grid = (BATCH // _BN,)
    return pl.pallas_call(
        _tc_body,
        grid=grid,
        in_specs=[
            pl.BlockSpec((EMBED_DIM, INPUT_DIM), lambda i: (0, 0)),
            pl.BlockSpec((_BN, INPUT_DIM), lambda i: (i, 0)),
            pl.BlockSpec((EMBED_DIM, 1), lambda i: (0, 0)),
            pl.BlockSpec((EMBED_DIM, _BN), lambda i: (0, i)),
        ],
        out_specs=pl.BlockSpec((EMBED_DIM, _BN), lambda i: (0, i)),
        out_shape=jax.ShapeDtypeStruct((EMBED_DIM, BATCH), jnp.float32),
    )(W, x, b, g_t)


@jax.jit
def kernel(x, domain_ids, domain_table, W, b):
    g_t = _sc_gather_t(domain_ids.astype(jnp.int32), domain_table.T)
    out_t = _tc_proj_add(W, x, b.reshape(EMBED_DIM, 1), g_t)
    return out_t.T
